# baseline (speedup 1.0000x reference)
import functools

import numpy as np

import jax
import jax.numpy as jnp
from jax import lax
from jax.experimental import pallas as pl
from jax.experimental.pallas import tpu as pltpu
from jax.experimental.pallas import tpu_sc as plsc

_NC, _NS = 2, 16
_NW = _NC * _NS
_N, _D, _H, _W, _C = 2, 64, 64, 64, 32
_LINES = _N * (_D // 2) * (_H // 2)
_LPW = _LINES // _NW
_ROWS_IN = _LINES * (_W // 4)
_CHUNK = 128
_NCHUNK = _ROWS_IN // _NW // _CHUNK
_LPC = _LPW // _NCHUNK
_OUT_ROWS = _N * (_D // 2) * (_H // 2) * (_W // 2) * _C // 128
_ORPC = _OUT_ROWS // _NW // _NCHUNK


def _axis_picks(key, dim, t):
    elem = t // 2
    nb = dim // t
    keys = jax.random.split(key, nb)
    perms = jax.vmap(lambda k: jax.random.permutation(k, t)[:elem])(keys)
    perms = jnp.sort(perms, axis=1)
    offsets = (jnp.arange(nb) * t)[:, None]
    return (perms + offsets).reshape(-1)


def _build_kernel(cc):
    moves = []
    for v in range(64):
        j = v // 2
        src_r, src_c = cc[j] // 4, (cc[j] % 4) * 32 + (v % 2) * 16
        dst_r, dst_c = v // 8, (16 * v) % 128
        moves.append((src_r, src_c, dst_r, dst_c))

    mesh = plsc.VectorSubcoreMesh(core_axis_name="c", subcore_axis_name="s")

    @functools.partial(
        pl.kernel,
        mesh=mesh,
        out_type=jax.ShapeDtypeStruct((_OUT_ROWS, 128), jnp.float32),
        scratch_types=[
            pltpu.VMEM((_NCHUNK, _CHUNK), jnp.int32),
            pltpu.VMEM((_CHUNK, 128), jnp.float32),
            pltpu.VMEM((_CHUNK, 128), jnp.float32),
            pltpu.VMEM((_ORPC, 128), jnp.float32),
            pltpu.VMEM((_ORPC, 128), jnp.float32),
            pltpu.SemaphoreType.DMA,
            pltpu.SemaphoreType.DMA,
        ],
    )
    def gather(table_hbm, idx_hbm, out_hbm, idx_v, in0, in1, ob0, ob1,
               gsem, osem):
        wid = lax.axis_index("s") * _NC + lax.axis_index("c")
        pltpu.sync_copy(idx_hbm.at[wid], idx_v)
        ins, obs = (in0, in1), (ob0, ob1)

        def compress(src, dst):
            def line(l, _):
                for sr, sc, dr, dc in moves:
                    dst[8 * l + dr, pl.ds(dc, 16)] = src[
                        16 * l + sr, pl.ds(sc, 16)
                    ]
                return _

            lax.fori_loop(0, _LPC, line, 0)

        gcp = [None] * _NCHUNK
        ocp = [None, None]
        gcp[0] = pltpu.async_copy(table_hbm.at[idx_v.at[0]], in0, gsem)
        for c in range(_NCHUNK):
            if c + 1 < _NCHUNK:
                gcp[c + 1] = pltpu.async_copy(
                    table_hbm.at[idx_v.at[c + 1]], ins[(c + 1) % 2], gsem
                )
            gcp[c].wait()
            if ocp[c % 2] is not None:
                ocp[c % 2].wait()
            compress(ins[c % 2], obs[c % 2])
            ocp[c % 2] = pltpu.async_copy(
                obs[c % 2],
                out_hbm.at[pl.ds(wid * (_ORPC * _NCHUNK) + c * _ORPC, _ORPC)],
                osem,
            )
        ocp[0].wait()
        ocp[1].wait()

    return gather


def kernel(inputs, t):
    del t
    with jax.ensure_compile_time_eval():
        base = jax.random.key(42)
        kz, kr, kc = jax.random.split(base, 3)
        c_z = _axis_picks(kz, _D, 4)
        c_rows = _axis_picks(kr, _H, 4)
        c_cols = _axis_picks(kc, _W, 4)
        n_ix = jnp.arange(_N, dtype=jnp.int32)
        lines = (
            n_ix[:, None, None] * _D + c_z[None, :, None]
        ) * _H + c_rows[None, None, :]
        src = lines.reshape(-1, 1) * (_W // 4) + jnp.arange(
            _W // 4, dtype=jnp.int32
        )
        idx = src.astype(jnp.int32).reshape(_NW, _NCHUNK, _CHUNK)
        cc_static = tuple(int(x) for x in np.asarray(c_cols))
    table = inputs.reshape(_N * _D * _H * _W // 4, 128)
    out = _build_kernel(cc_static)(table, idx)
    return out.reshape(_N, _D // 2, _H // 2, _W // 2, _C)


# speedup vs baseline: 1.4319x; 1.4319x over previous
import functools

import numpy as np

import jax
import jax.numpy as jnp
from jax import lax
from jax.experimental import pallas as pl
from jax.experimental.pallas import tpu as pltpu
from jax.experimental.pallas import tpu_sc as plsc

_NC, _NS = 2, 16
_NW = _NC * _NS
_N, _D, _H, _W, _C = 2, 64, 64, 64, 32
_SLABS = _N * (_D // 2) * (_H // 2)
_SPW = _SLABS // _NW
_SPC = 8
_NCHUNK = _SPW // _SPC

_CZ = (2, 3, 5, 6, 8, 9, 13, 14, 17, 18, 22, 23, 24, 25, 30, 31,
       33, 34, 38, 39, 41, 42, 45, 47, 48, 51, 53, 54, 56, 57, 60, 61)
_CR = (1, 3, 5, 6, 8, 9, 13, 14, 17, 18, 21, 23, 24, 25, 29, 30,
       32, 34, 36, 39, 42, 43, 45, 47, 48, 49, 52, 54, 56, 59, 60, 63)
_CC = (1, 2, 6, 7, 9, 10, 14, 15, 16, 17, 22, 23, 24, 26, 28, 29,
       34, 35, 37, 39, 40, 42, 44, 46, 48, 49, 54, 55, 57, 59, 60, 62)


@functools.cache
def _gather_kernel():
    mesh = plsc.VectorSubcoreMesh(core_axis_name="c", subcore_axis_name="s")

    @functools.partial(
        pl.kernel,
        mesh=mesh,
        compiler_params=pltpu.CompilerParams(
            use_tc_tiling_on_sc=False, needs_layout_passes=False
        ),
        out_type=jax.ShapeDtypeStruct((_SLABS, _C * _W // 2), jnp.float32),
        scratch_types=[
            pltpu.VMEM((_SPW,), jnp.int32),
            pltpu.VMEM((_SPC, _C * _W), jnp.float32),
            pltpu.VMEM((_SPC, _C * _W), jnp.float32),
            pltpu.VMEM((_SPC, _C * _W // 2), jnp.float32),
            pltpu.VMEM((_SPC, _C * _W // 2), jnp.float32),
            pltpu.SemaphoreType.DMA,
            pltpu.SemaphoreType.DMA,
        ],
    )
    def gather(table_hbm, idx_hbm, out_hbm, idx_v,
               in0, in1, ob0, ob1, gsem, osem):
        wid = lax.axis_index("s") * _NC + lax.axis_index("c")
        pltpu.sync_copy(idx_hbm.at[wid], idx_v)
        ins, obs = (in0, in1), (ob0, ob1)
        iota_w = lax.iota(jnp.int32, 16) * _W

        def compress(src, dst):
            def slab(s, _):
                srow = src.at[s]
                for j in range(_W // 2):
                    for h in range(2):
                        dst[s, pl.ds(j * _C + h * 16, 16)] = plsc.load_gather(
                            srow, [iota_w + (h * 16 * _W + _CC[j])]
                        )
                return _

            lax.fori_loop(0, _SPC, slab, 0)

        gcp = [None] * _NCHUNK
        ocp = [None, None]
        gcp[0] = pltpu.async_copy(
            table_hbm.at[idx_v.at[pl.ds(0, _SPC)]], in0, gsem
        )
        for c in range(_NCHUNK):
            if c + 1 < _NCHUNK:
                gcp[c + 1] = pltpu.async_copy(
                    table_hbm.at[idx_v.at[pl.ds((c + 1) * _SPC, _SPC)]],
                    ins[(c + 1) % 2],
                    gsem,
                )
            gcp[c].wait()
            if ocp[c % 2] is not None:
                ocp[c % 2].wait()
            compress(ins[c % 2], obs[c % 2])
            ocp[c % 2] = pltpu.async_copy(
                obs[c % 2],
                out_hbm.at[pl.ds(wid * _SPW + c * _SPC, _SPC)],
                osem,
            )
        ocp[0].wait()
        ocp[1].wait()

    return gather


def kernel(inputs, t):
    del t
    cz = np.asarray(_CZ, np.int32)
    cr = np.asarray(_CR, np.int32)
    n_ix = np.arange(_N, dtype=np.int32)
    slab = (n_ix[:, None, None] * _D + cz[:, None]) * _H + cr
    idx = jnp.asarray(slab.reshape(_NW, _SPW))
    table = jnp.transpose(inputs, (0, 1, 2, 4, 3)).reshape(
        _N * _D * _H, _C * _W
    )
    out = _gather_kernel()(table, idx)
    return out.reshape(_N, _D // 2, _H // 2, _W // 2, _C)


# speedup vs baseline: 3.7704x; 2.6332x over previous
import functools

import numpy as np

import jax
import jax.numpy as jnp
from jax import lax
from jax.experimental import pallas as pl
from jax.experimental.pallas import tpu as pltpu
from jax.experimental.pallas import tpu_sc as plsc

_NC, _NS = 2, 16
_NW = _NC * _NS
_N, _D, _H, _W, _C = 2, 64, 64, 64, 32
_SLABS = _N * (_D // 2) * (_H // 2)
_SPW = _SLABS // _NW
_RND = 4
_NROUND = _SPW // _RND

_CZ = (2, 3, 5, 6, 8, 9, 13, 14, 17, 18, 22, 23, 24, 25, 30, 31,
       33, 34, 38, 39, 41, 42, 45, 47, 48, 51, 53, 54, 56, 57, 60, 61)
_CR = (1, 3, 5, 6, 8, 9, 13, 14, 17, 18, 21, 23, 24, 25, 29, 30,
       32, 34, 36, 39, 42, 43, 45, 47, 48, 49, 52, 54, 56, 59, 60, 63)
_CC = (1, 2, 6, 7, 9, 10, 14, 15, 16, 17, 22, 23, 24, 26, 28, 29,
       34, 35, 37, 39, 40, 42, 44, 46, 48, 49, 54, 55, 57, 59, 60, 62)


@functools.cache
def _gather_kernel():
    mesh = plsc.VectorSubcoreMesh(core_axis_name="c", subcore_axis_name="s")

    @functools.partial(
        pl.kernel,
        mesh=mesh,
        compiler_params=pltpu.CompilerParams(needs_layout_passes=False),
        out_type=jax.ShapeDtypeStruct((_SLABS, _W // 2, _C), jnp.float32),
        scratch_types=(
            [pltpu.VMEM((_SPW,), jnp.int32)]
            + [pltpu.VMEM((1, _C, _W), jnp.float32) for _ in range(2 * _RND)]
            + [pltpu.VMEM((1, _W // 2, _C), jnp.float32) for _ in range(2 * _RND)]
            + [pltpu.SemaphoreType.DMA for _ in range(4)]
        ),
    )
    def gather(table_hbm, idx_hbm, out_hbm, idx_v, *bufs):
        ina = bufs[0:_RND]
        inb = bufs[_RND:2 * _RND]
        outa = bufs[2 * _RND:3 * _RND]
        outb = bufs[3 * _RND:4 * _RND]
        gsa, gsb, osa, osb = bufs[4 * _RND:]
        wid = lax.axis_index("s") * _NC + lax.axis_index("c")
        pltpu.sync_copy(idx_hbm.at[wid], idx_v)
        iota16 = lax.iota(jnp.int32, 16)

        def slab_id(q):
            vec = idx_v[pl.ds((q // 16) * 16, 16)]
            return jnp.sum(jnp.where(iota16 == q % 16, vec, 0))

        def issue_in(q, buf, sem):
            pltpu.async_copy(table_hbm.at[pl.ds(slab_id(q), 1)], buf, sem)

        def drain(dummy_src, buf, sem):
            pltpu.make_async_copy(dummy_src, buf, sem).wait()

        def compress(src, dst):
            s2 = src.at[0]
            for j in range(_W // 2):
                cc_j = jnp.broadcast_to(jnp.int32(_CC[j]), (16,))
                for h in range(2):
                    dst[0, j, pl.ds(h * 16, 16)] = plsc.load_gather(
                        s2, [iota16 + h * 16, cc_j]
                    )

        def round_(t, base, nxt_exists, ins, nxt_ins, outs, gsem, nxt_gsem,
                   osem):
            @pl.when(nxt_exists)
            def _():
                for b in range(_RND):
                    issue_in(base + _RND + b, nxt_ins[b], nxt_gsem)

            for b in range(_RND):
                drain(table_hbm.at[pl.ds(0, 1)], ins[b], gsem)

            @pl.when(t > 0)
            def _():
                for b in range(_RND):
                    drain(out_hbm.at[pl.ds(0, 1)], outs[b], osem)

            for b in range(_RND):
                compress(ins[b], outs[b])
                pltpu.async_copy(
                    outs[b], out_hbm.at[pl.ds(wid * _SPW + base + b, 1)], osem
                )

        for b in range(_RND):
            issue_in(b, ina[b], gsa)

        def body(t, carry):
            base = 2 * _RND * t
            round_(t, base, base + _RND < _SPW, ina, inb, outa, gsa, gsb, osa)
            round_(t, base + _RND, base + 2 * _RND < _SPW, inb, ina, outb,
                   gsb, gsa, osb)
            return carry

        lax.fori_loop(0, _NROUND // 2, body, 0)
        for b in range(_RND):
            drain(out_hbm.at[pl.ds(0, 1)], outa[b], osa)
            drain(out_hbm.at[pl.ds(0, 1)], outb[b], osb)

    return gather


def kernel(inputs, t):
    del t
    cz = np.asarray(_CZ, np.int32)
    cr = np.asarray(_CR, np.int32)
    n_ix = np.arange(_N, dtype=np.int32)
    slab = (n_ix[:, None, None] * _D + cz[:, None]) * _H + cr
    idx = jnp.asarray(slab.reshape(_NW, _SPW))
    table = jnp.transpose(inputs, (0, 1, 2, 4, 3)).reshape(
        _N * _D * _H, _C, _W
    )
    out = _gather_kernel()(table, idx)
    return out.reshape(_N, _D // 2, _H // 2, _W // 2, _C)


# speedup vs baseline: 4.8540x; 1.2874x over previous
import functools

import numpy as np

import jax
import jax.numpy as jnp
from jax import lax
from jax.experimental import pallas as pl
from jax.experimental.pallas import tpu as pltpu
from jax.experimental.pallas import tpu_sc as plsc

_NC, _NS = 2, 16
_NW = _NC * _NS
_N, _D, _H, _W, _C = 2, 64, 64, 64, 32
_SLABS = _N * (_D // 2) * (_H // 2)
_SPW = _SLABS // _NW
_RND = 4
_NROUND = _SPW // _RND

_CZ = (2, 3, 5, 6, 8, 9, 13, 14, 17, 18, 22, 23, 24, 25, 30, 31,
       33, 34, 38, 39, 41, 42, 45, 47, 48, 51, 53, 54, 56, 57, 60, 61)
_CR = (1, 3, 5, 6, 8, 9, 13, 14, 17, 18, 21, 23, 24, 25, 29, 30,
       32, 34, 36, 39, 42, 43, 45, 47, 48, 49, 52, 54, 56, 59, 60, 63)
_CC = (1, 2, 6, 7, 9, 10, 14, 15, 16, 17, 22, 23, 24, 26, 28, 29,
       34, 35, 37, 39, 40, 42, 44, 46, 48, 49, 54, 55, 57, 59, 60, 62)


@functools.cache
def _gather_kernel():
    mesh = plsc.VectorSubcoreMesh(core_axis_name="c", subcore_axis_name="s")

    @functools.partial(
        pl.kernel,
        mesh=mesh,
        compiler_params=pltpu.CompilerParams(needs_layout_passes=False),
        out_type=jax.ShapeDtypeStruct((_SLABS, _W // 2, _C), jnp.float32),
        scratch_types=(
            [pltpu.VMEM((_SPW,), jnp.int32)]
            + [pltpu.VMEM((1, _C, _W), jnp.float32) for _ in range(2 * _RND)]
            + [pltpu.VMEM((1, _W // 2, _C), jnp.float32) for _ in range(2 * _RND)]
            + [pltpu.SemaphoreType.DMA for _ in range(4)]
        ),
    )
    def gather(table_hbm, idx_hbm, out_hbm, idx_v, *bufs):
        ina = bufs[0:_RND]
        inb = bufs[_RND:2 * _RND]
        outa = bufs[2 * _RND:3 * _RND]
        outb = bufs[3 * _RND:4 * _RND]
        gsa, gsb, osa, osb = bufs[4 * _RND:]
        wid = lax.axis_index("s") * _NC + lax.axis_index("c")
        pltpu.sync_copy(idx_hbm.at[wid], idx_v)
        iota16 = lax.iota(jnp.int32, 16)

        def slab_id(q):
            vec = idx_v[pl.ds((q // 16) * 16, 16)]
            return jnp.sum(jnp.where(iota16 == q % 16, vec, 0))

        def issue_in(q, buf, sem):
            pltpu.async_copy(table_hbm.at[pl.ds(slab_id(q), 1)], buf, sem)

        def drain(dummy_src, buf, sem):
            pltpu.make_async_copy(dummy_src, buf, sem).wait()

        iotas = (iota16, iota16 + 16)

        def compress(src, dst):
            s2 = src.at[0]
            depth = 4
            pending = []
            for j in range(_W // 2):
                cc_j = jnp.broadcast_to(jnp.int32(_CC[j]), (16,))
                vals = tuple(
                    plsc.load_gather(s2, [iotas[h], cc_j]) for h in range(2)
                )
                pending.append((j, vals))
                if len(pending) >= depth:
                    pj, pv = pending.pop(0)
                    dst[0, pj, pl.ds(0, 16)] = pv[0]
                    dst[0, pj, pl.ds(16, 16)] = pv[1]
            for pj, pv in pending:
                dst[0, pj, pl.ds(0, 16)] = pv[0]
                dst[0, pj, pl.ds(16, 16)] = pv[1]

        def round_(t, base, nxt_exists, ins, nxt_ins, outs, gsem, nxt_gsem,
                   osem):
            @pl.when(nxt_exists)
            def _():
                for b in range(_RND):
                    issue_in(base + _RND + b, nxt_ins[b], nxt_gsem)

            for b in range(_RND):
                drain(table_hbm.at[pl.ds(0, 1)], ins[b], gsem)

            @pl.when(t > 0)
            def _():
                for b in range(_RND):
                    drain(out_hbm.at[pl.ds(0, 1)], outs[b], osem)

            for b in range(_RND):
                compress(ins[b], outs[b])
                pltpu.async_copy(
                    outs[b], out_hbm.at[pl.ds(wid * _SPW + base + b, 1)], osem
                )

        for b in range(_RND):
            issue_in(b, ina[b], gsa)

        def body(t, carry):
            base = 2 * _RND * t
            round_(t, base, base + _RND < _SPW, ina, inb, outa, gsa, gsb, osa)
            round_(t, base + _RND, base + 2 * _RND < _SPW, inb, ina, outb,
                   gsb, gsa, osb)
            return carry

        lax.fori_loop(0, _NROUND // 2, body, 0)
        for b in range(_RND):
            drain(out_hbm.at[pl.ds(0, 1)], outa[b], osa)
            drain(out_hbm.at[pl.ds(0, 1)], outb[b], osb)

    return gather


def kernel(inputs, t):
    del t
    cz = np.asarray(_CZ, np.int32)
    cr = np.asarray(_CR, np.int32)
    n_ix = np.arange(_N, dtype=np.int32)
    slab = (n_ix[:, None, None] * _D + cz[:, None]) * _H + cr
    idx = jnp.asarray(slab.reshape(_NW, _SPW))
    table = jnp.transpose(inputs, (0, 1, 2, 4, 3)).reshape(
        _N * _D * _H, _C, _W
    )
    out = _gather_kernel()(table, idx)
    return out.reshape(_N, _D // 2, _H // 2, _W // 2, _C)
